# Initial kernel scaffold; baseline (speedup 1.0000x reference)
import jax
import jax.numpy as jnp
from jax.experimental import pallas as pl


def kernel(x, edge_index, batch, W1, b1, W2, b2, lin_W, lin_b, out_W, out_b):
    raise NotImplementedError("write your pallas kernel here")


# speedup vs baseline: 15.7381x; 15.7381x over previous
import dataclasses
import functools

import jax
import jax.numpy as jnp
from jax import lax
from jax.experimental import pallas as pl
from jax.experimental.pallas import tpu as pltpu
from jax.experimental.pallas import tpu_sc as plsc

N = 10000
E = 320000
D = 128
G = 64
LIN = 128
OUT = 8

K = 128
NCHUNK = E // K
NTILES = 32
RPT = 624
RTAIL = N - 16 * RPT
HR = 640

_mesh = plsc.VectorSubcoreMesh(core_axis_name="c", subcore_axis_name="s")

_sc_params = pltpu.CompilerParams()
if "needs_layout_passes" in pltpu.CompilerParams.__dataclass_fields__:
    _sc_params = dataclasses.replace(_sc_params, needs_layout_passes=False)


@functools.partial(
    pl.kernel,
    out_type=jax.ShapeDtypeStruct((2, HR, D), jnp.float32),
    mesh=_mesh,
    scratch_types=[
        pltpu.VMEM((K,), jnp.int32),
        pltpu.VMEM((HR, D), jnp.float32),
        pltpu.VMEM((K,), jnp.int32),
        pltpu.VMEM_SHARED((HR, D), jnp.float32),
    ],
    compiler_params=_sc_params,
)
def _sc_hist(dst_hbm, zeros_hbm, out_hbm, didx, hist, iota_v, acc):
    core = lax.axis_index("c")
    sid = lax.axis_index("s")
    wid = sid * 2 + core
    hrt = HR // 16
    r0 = sid * hrt
    pltpu.sync_copy(zeros_hbm, hist)
    pltpu.sync_copy(zeros_hbm.at[pl.ds(r0, hrt)], acc.at[pl.ds(r0, hrt)])
    base16 = lax.iota(jnp.int32, 16)
    one16 = jnp.full((16,), 1.0, jnp.float32)

    @pl.loop(0, (NCHUNK + NTILES - 1) // NTILES)
    def _(i):
        c = i * NTILES + wid

        @pl.when(c < NCHUNK)
        def _():
            pltpu.sync_copy(dst_hbm.at[c], didx)
            for j in range(K // 16):
                d16 = didx[pl.ds(j * 16, 16)]
                row = lax.shift_right_logical(d16, 7)
                col = lax.bitwise_and(d16, 127)
                plsc.addupdate_scatter(hist, [row, col], one16)

    plsc.subcore_barrier()

    for k in range(HR // K):
        for j in range(K // 16):
            iota_v[pl.ds(j * 16, 16)] = base16 + (k * K + j * 16)
        pltpu.sync_copy(hist.at[pl.ds(k * K, K)], acc.at[iota_v], add=True)

    plsc.subcore_barrier()
    pltpu.sync_copy(acc.at[pl.ds(r0, hrt)], out_hbm.at[core].at[pl.ds(r0, hrt)])


@functools.partial(
    pl.kernel,
    out_type=jax.ShapeDtypeStruct((2, N, D), jnp.float32),
    mesh=_mesh,
    scratch_types=[
        pltpu.VMEM((K,), jnp.int32),
        pltpu.VMEM((K,), jnp.int32),
        pltpu.VMEM((K, D), jnp.float32),
        pltpu.VMEM_SHARED((N, D), jnp.float32),
        pltpu.SemaphoreType.DMA,
    ],
)
def _sc_agg(y_hbm, zeros_hbm, src_hbm, dst_hbm, out_hbm, sidx, didx, rows, acc, sem):
    core = lax.axis_index("c")
    sid = lax.axis_index("s")
    wid = sid * 2 + core
    r0 = sid * RPT

    @pl.when(core == 0)
    def _():
        pltpu.sync_copy(y_hbm.at[pl.ds(r0, RPT)], acc.at[pl.ds(r0, RPT)])

        @pl.when(sid == 15)
        def _():
            pltpu.sync_copy(y_hbm.at[pl.ds(16 * RPT, RTAIL)],
                            acc.at[pl.ds(16 * RPT, RTAIL)])

    @pl.when(core == 1)
    def _():
        pltpu.sync_copy(zeros_hbm.at[pl.ds(r0, RPT)], acc.at[pl.ds(r0, RPT)])

        @pl.when(sid == 15)
        def _():
            pltpu.sync_copy(zeros_hbm.at[pl.ds(16 * RPT, RTAIL)],
                            acc.at[pl.ds(16 * RPT, RTAIL)])

    plsc.subcore_barrier()

    @pl.loop(0, (NCHUNK + NTILES - 1) // NTILES)
    def _(i):
        c = i * NTILES + wid

        @pl.when(c < NCHUNK)
        def _():
            pltpu.sync_copy(src_hbm.at[c], sidx)
            pltpu.sync_copy(dst_hbm.at[c], didx)
            pltpu.async_copy(y_hbm.at[sidx], rows, sem).wait()
            pltpu.sync_copy(rows, acc.at[didx], add=True)

    plsc.subcore_barrier()
    pltpu.sync_copy(acc.at[pl.ds(r0, RPT)], out_hbm.at[core].at[pl.ds(r0, RPT)])

    @pl.when(sid == 15)
    def _():
        pltpu.sync_copy(acc.at[pl.ds(16 * RPT, RTAIL)],
                        out_hbm.at[core].at[pl.ds(16 * RPT, RTAIL)])


R = 1000
GRID = N // R


def _tdinv_body(degp_ref, dinv_ref):
    dinv_ref[...] = lax.rsqrt(1.0 + degp_ref[0] + degp_ref[1])


_tdinv = pl.pallas_call(
    _tdinv_body,
    grid=(1,),
    in_specs=[pl.BlockSpec((2, HR, D), lambda i: (0, 0, 0))],
    out_specs=pl.BlockSpec((HR, D), lambda i: (0, 0)),
    out_shape=jax.ShapeDtypeStruct((HR, D), jnp.float32),
)


def _t1_body(x_ref, w_ref, dinv_ref, y_ref):
    dinv = dinv_ref[...]
    xw = jnp.dot(x_ref[...], w_ref[...], preferred_element_type=jnp.float32)
    y_ref[...] = xw * dinv


_t1 = pl.pallas_call(
    _t1_body,
    grid=(GRID,),
    in_specs=[
        pl.BlockSpec((R, D), lambda i: (i, 0)),
        pl.BlockSpec((D, D), lambda i: (0, 0)),
        pl.BlockSpec((R, 1), lambda i: (i, 0)),
    ],
    out_specs=pl.BlockSpec((R, D), lambda i: (i, 0)),
    out_shape=jax.ShapeDtypeStruct((N, D), jnp.float32),
)


def _t2_body(agg_ref, dinv_ref, b1_ref, w2_ref, y2_ref):
    dinv = dinv_ref[...]
    a = agg_ref[0] + agg_ref[1]
    h = jnp.maximum(a * dinv + b1_ref[...], 0.0)
    y2_ref[...] = jnp.dot(h, w2_ref[...], preferred_element_type=jnp.float32) * dinv


_t2 = pl.pallas_call(
    _t2_body,
    grid=(GRID,),
    in_specs=[
        pl.BlockSpec((2, R, D), lambda i: (0, i, 0)),
        pl.BlockSpec((R, 1), lambda i: (i, 0)),
        pl.BlockSpec((1, D), lambda i: (0, 0)),
        pl.BlockSpec((D, D), lambda i: (0, 0)),
    ],
    out_specs=pl.BlockSpec((R, D), lambda i: (i, 0)),
    out_shape=jax.ShapeDtypeStruct((N, D), jnp.float32),
)


def _t3_body(agg_ref, dinv_ref, b2_ref, batch_ref, lw_ref, lb_ref, ow_ref,
             ob_ref, out_ref, pool_ref):
    i = pl.program_id(0)

    @pl.when(i == 0)
    def _():
        pool_ref[...] = jnp.full((G, D), -jnp.inf, jnp.float32)

    dinv = dinv_ref[...]
    a = agg_ref[0] + agg_ref[1]
    h = jnp.maximum(a * dinv + b2_ref[...], 0.0)
    bidx = batch_ref[...]

    g_lo = bidx[0, 0]
    g_hi = bidx[R - 1, 0]

    def body(g, carry):
        vals = jnp.where(bidx == g, h, -jnp.inf)
        cur = pool_ref[pl.ds(g, 1), :]
        pool_ref[pl.ds(g, 1), :] = jnp.maximum(cur, vals.max(axis=0, keepdims=True))
        return carry

    lax.fori_loop(g_lo, g_hi + 1, body, 0)

    @pl.when(i == GRID - 1)
    def _():
        t = jnp.dot(pool_ref[...], lw_ref[...],
                    preferred_element_type=jnp.float32) + lb_ref[...]
        out_ref[...] = jnp.dot(t, ow_ref[...],
                               preferred_element_type=jnp.float32) + ob_ref[...]


_t3 = pl.pallas_call(
    _t3_body,
    grid=(GRID,),
    in_specs=[
        pl.BlockSpec((2, R, D), lambda i: (0, i, 0)),
        pl.BlockSpec((R, 1), lambda i: (i, 0)),
        pl.BlockSpec((1, D), lambda i: (0, 0)),
        pl.BlockSpec((R, 1), lambda i: (i, 0)),
        pl.BlockSpec((D, LIN), lambda i: (0, 0)),
        pl.BlockSpec((1, LIN), lambda i: (0, 0)),
        pl.BlockSpec((LIN, OUT), lambda i: (0, 0)),
        pl.BlockSpec((1, OUT), lambda i: (0, 0)),
    ],
    out_specs=pl.BlockSpec((G, OUT), lambda i: (0, 0)),
    out_shape=jax.ShapeDtypeStruct((G, OUT), jnp.float32),
    scratch_shapes=[pltpu.VMEM((G, D), jnp.float32)],
)


def kernel(x, edge_index, batch, W1, b1, W2, b2, lin_W, lin_b, out_W, out_b):
    src2 = edge_index[0].reshape(NCHUNK, K)
    dst2 = edge_index[1].reshape(NCHUNK, K)
    zeros_nd = jnp.zeros((N, D), jnp.float32)
    zeros_hd = jnp.zeros((HR, D), jnp.float32)

    degp = _sc_hist(dst2, zeros_hd)
    dinv = _tdinv(degp).reshape(HR * D)[:N].reshape(N, 1)
    y1 = _t1(x, W1, dinv)
    agg1 = _sc_agg(y1, zeros_nd, src2, dst2)
    y2 = _t2(agg1, dinv, b1.reshape(1, D), W2)
    agg2 = _sc_agg(y2, zeros_nd, src2, dst2)
    return _t3(agg2, dinv, b2.reshape(1, D), batch.reshape(N, 1),
               lin_W, lin_b.reshape(1, LIN), out_W, out_b.reshape(1, OUT))


# speedup vs baseline: 21.1963x; 1.3468x over previous
import dataclasses
import functools

import jax
import jax.numpy as jnp
from jax import lax
from jax.experimental import pallas as pl
from jax.experimental.pallas import tpu as pltpu
from jax.experimental.pallas import tpu_sc as plsc

N = 10000
E = 320000
D = 128
G = 64
LIN = 128
OUT = 8

K = 128
NCHUNK = E // K
CPT = 80
HCPT = 40
NCPAD = 32 * CPT
NCLAST = NCHUNK - 31 * CPT
NTILES = 32
RPT = 624
RTAIL = N - 16 * RPT
HR = 640

_mesh = plsc.VectorSubcoreMesh(core_axis_name="c", subcore_axis_name="s")

_sc_params = pltpu.CompilerParams()
if "needs_layout_passes" in pltpu.CompilerParams.__dataclass_fields__:
    _sc_params = dataclasses.replace(_sc_params, needs_layout_passes=False)


@functools.partial(
    pl.kernel,
    out_type=jax.ShapeDtypeStruct((2, HR, D), jnp.float32),
    mesh=_mesh,
    scratch_types=[
        pltpu.VMEM((CPT, K), jnp.int32),
        pltpu.VMEM((HR, D), jnp.float32),
        pltpu.VMEM((K,), jnp.int32),
        pltpu.VMEM_SHARED((HR, D), jnp.float32),
    ],
    compiler_params=_sc_params,
)
def _sc_hist(dst_hbm, zeros_hbm, out_hbm, dstb, hist, iota_v, acc):
    core = lax.axis_index("c")
    sid = lax.axis_index("s")
    wid = sid * 2 + core
    nc = lax.select(wid == 31, NCLAST, CPT)
    hrt = HR // 16
    r0 = sid * hrt
    pltpu.sync_copy(dst_hbm.at[pl.ds(wid * CPT, CPT)], dstb)
    pltpu.sync_copy(zeros_hbm, hist)
    pltpu.sync_copy(zeros_hbm.at[pl.ds(r0, hrt)], acc.at[pl.ds(r0, hrt)])
    base16 = lax.iota(jnp.int32, 16)
    one16 = jnp.full((16,), 1.0, jnp.float32)

    @pl.loop(0, nc)
    def _(i):
        for j in range(K // 16):
            d16 = dstb[i, pl.ds(j * 16, 16)]
            row = lax.shift_right_logical(d16, 7)
            col = lax.bitwise_and(d16, 127)
            plsc.addupdate_scatter(hist, [row, col], one16)

    plsc.subcore_barrier()

    for k in range(HR // K):
        for j in range(K // 16):
            iota_v[pl.ds(j * 16, 16)] = base16 + (k * K + j * 16)
        pltpu.sync_copy(hist.at[pl.ds(k * K, K)], acc.at[iota_v], add=True)

    plsc.subcore_barrier()
    pltpu.sync_copy(acc.at[pl.ds(r0, hrt)], out_hbm.at[core].at[pl.ds(r0, hrt)])


@functools.partial(
    pl.kernel,
    out_type=jax.ShapeDtypeStruct((2, N, D), jnp.float32),
    mesh=_mesh,
    scratch_types=[
        pltpu.VMEM((HCPT, K), jnp.int32),
        pltpu.VMEM((HCPT, K), jnp.int32),
        pltpu.VMEM((K, D), jnp.float32),
        pltpu.VMEM((K, D), jnp.float32),
        pltpu.VMEM((K,), jnp.int32),
        pltpu.VMEM_SHARED((N, D), jnp.float32),
        pltpu.SemaphoreType.DMA,
        pltpu.SemaphoreType.DMA,
    ],
)
def _sc_agg(y_hbm, zeros_hbm, src_hbm, dst_hbm, out_hbm, srcb, dstb,
            rows0, rows1, didx, acc, gsem0, gsem1):
    core = lax.axis_index("c")
    sid = lax.axis_index("s")
    wid = sid * 2 + core
    nc = lax.select(wid == 31, NCLAST, CPT)
    r0 = sid * RPT

    @pl.when(core == 0)
    def _():
        pltpu.sync_copy(y_hbm.at[pl.ds(r0, RPT)], acc.at[pl.ds(r0, RPT)])

        @pl.when(sid == 15)
        def _():
            pltpu.sync_copy(y_hbm.at[pl.ds(16 * RPT, RTAIL)],
                            acc.at[pl.ds(16 * RPT, RTAIL)])

    @pl.when(core == 1)
    def _():
        pltpu.sync_copy(zeros_hbm.at[pl.ds(r0, RPT)], acc.at[pl.ds(r0, RPT)])

        @pl.when(sid == 15)
        def _():
            pltpu.sync_copy(zeros_hbm.at[pl.ds(16 * RPT, RTAIL)],
                            acc.at[pl.ds(16 * RPT, RTAIL)])

    plsc.subcore_barrier()

    for p in range(CPT // HCPT):
        ncp = jnp.clip(nc - p * HCPT, 0, HCPT)

        @pl.when(ncp > 0)
        def _():
            pltpu.sync_copy(src_hbm.at[pl.ds(wid * CPT + p * HCPT, HCPT)], srcb)
            pltpu.sync_copy(dst_hbm.at[pl.ds(wid * CPT + p * HCPT, HCPT)], dstb)

            @pl.loop(0, ncp)
            def _(i):
                pltpu.sync_copy(y_hbm.at[srcb.at[i]], rows0)
                for j in range(K // 16):
                    didx[pl.ds(j * 16, 16)] = dstb[i, pl.ds(j * 16, 16)]
                pltpu.sync_copy(rows0, acc.at[didx], add=True)

    plsc.subcore_barrier()
    pltpu.sync_copy(acc.at[pl.ds(r0, RPT)], out_hbm.at[core].at[pl.ds(r0, RPT)])

    @pl.when(sid == 15)
    def _():
        pltpu.sync_copy(acc.at[pl.ds(16 * RPT, RTAIL)],
                        out_hbm.at[core].at[pl.ds(16 * RPT, RTAIL)])


R = 1000
GRID = N // R


def _tdinv_body(degp_ref, dinv_ref):
    dinv_ref[...] = lax.rsqrt(1.0 + degp_ref[0] + degp_ref[1])


_tdinv = pl.pallas_call(
    _tdinv_body,
    grid=(1,),
    in_specs=[pl.BlockSpec((2, HR, D), lambda i: (0, 0, 0))],
    out_specs=pl.BlockSpec((HR, D), lambda i: (0, 0)),
    out_shape=jax.ShapeDtypeStruct((HR, D), jnp.float32),
)


def _t1_body(x_ref, w_ref, dinv_ref, y_ref):
    dinv = dinv_ref[...]
    xw = jnp.dot(x_ref[...], w_ref[...], preferred_element_type=jnp.float32)
    y_ref[...] = xw * dinv


_t1 = pl.pallas_call(
    _t1_body,
    grid=(GRID,),
    in_specs=[
        pl.BlockSpec((R, D), lambda i: (i, 0)),
        pl.BlockSpec((D, D), lambda i: (0, 0)),
        pl.BlockSpec((R, 1), lambda i: (i, 0)),
    ],
    out_specs=pl.BlockSpec((R, D), lambda i: (i, 0)),
    out_shape=jax.ShapeDtypeStruct((N, D), jnp.float32),
)


def _t2_body(agg_ref, dinv_ref, b1_ref, w2_ref, y2_ref):
    dinv = dinv_ref[...]
    a = agg_ref[0] + agg_ref[1]
    h = jnp.maximum(a * dinv + b1_ref[...], 0.0)
    y2_ref[...] = jnp.dot(h, w2_ref[...], preferred_element_type=jnp.float32) * dinv


_t2 = pl.pallas_call(
    _t2_body,
    grid=(GRID,),
    in_specs=[
        pl.BlockSpec((2, R, D), lambda i: (0, i, 0)),
        pl.BlockSpec((R, 1), lambda i: (i, 0)),
        pl.BlockSpec((1, D), lambda i: (0, 0)),
        pl.BlockSpec((D, D), lambda i: (0, 0)),
    ],
    out_specs=pl.BlockSpec((R, D), lambda i: (i, 0)),
    out_shape=jax.ShapeDtypeStruct((N, D), jnp.float32),
)


def _t3_body(agg_ref, dinv_ref, b2_ref, batch_ref, lw_ref, lb_ref, ow_ref,
             ob_ref, out_ref, pool_ref):
    i = pl.program_id(0)

    @pl.when(i == 0)
    def _():
        pool_ref[...] = jnp.full((G, D), -jnp.inf, jnp.float32)

    dinv = dinv_ref[...]
    a = agg_ref[0] + agg_ref[1]
    h = jnp.maximum(a * dinv + b2_ref[...], 0.0)
    bidx = batch_ref[...]

    g_lo = bidx[0, 0]
    g_hi = bidx[R - 1, 0]

    def body(g, carry):
        vals = jnp.where(bidx == g, h, -jnp.inf)
        cur = pool_ref[pl.ds(g, 1), :]
        pool_ref[pl.ds(g, 1), :] = jnp.maximum(cur, vals.max(axis=0, keepdims=True))
        return carry

    lax.fori_loop(g_lo, g_hi + 1, body, 0)

    @pl.when(i == GRID - 1)
    def _():
        t = jnp.dot(pool_ref[...], lw_ref[...],
                    preferred_element_type=jnp.float32) + lb_ref[...]
        out_ref[...] = jnp.dot(t, ow_ref[...],
                               preferred_element_type=jnp.float32) + ob_ref[...]


_t3 = pl.pallas_call(
    _t3_body,
    grid=(GRID,),
    in_specs=[
        pl.BlockSpec((2, R, D), lambda i: (0, i, 0)),
        pl.BlockSpec((R, 1), lambda i: (i, 0)),
        pl.BlockSpec((1, D), lambda i: (0, 0)),
        pl.BlockSpec((R, 1), lambda i: (i, 0)),
        pl.BlockSpec((D, LIN), lambda i: (0, 0)),
        pl.BlockSpec((1, LIN), lambda i: (0, 0)),
        pl.BlockSpec((LIN, OUT), lambda i: (0, 0)),
        pl.BlockSpec((1, OUT), lambda i: (0, 0)),
    ],
    out_specs=pl.BlockSpec((G, OUT), lambda i: (0, 0)),
    out_shape=jax.ShapeDtypeStruct((G, OUT), jnp.float32),
    scratch_shapes=[pltpu.VMEM((G, D), jnp.float32)],
)


def kernel(x, edge_index, batch, W1, b1, W2, b2, lin_W, lin_b, out_W, out_b):
    pad = jnp.zeros((NCPAD - NCHUNK, K), jnp.int32)
    src2 = jnp.concatenate([edge_index[0].reshape(NCHUNK, K), pad])
    dst2 = jnp.concatenate([edge_index[1].reshape(NCHUNK, K), pad])
    zeros_nd = jnp.zeros((N, D), jnp.float32)
    zeros_hd = jnp.zeros((HR, D), jnp.float32)

    degp = _sc_hist(dst2, zeros_hd)
    dinv = _tdinv(degp).reshape(HR * D)[:N].reshape(N, 1)
    y1 = _t1(x, W1, dinv)
    agg1 = _sc_agg(y1, zeros_nd, src2, dst2)
    y2 = _t2(agg1, dinv, b1.reshape(1, D), W2)
    agg2 = _sc_agg(y2, zeros_nd, src2, dst2)
    return _t3(agg2, dinv, b2.reshape(1, D), batch.reshape(N, 1),
               lin_W, lin_b.reshape(1, LIN), out_W, out_b.reshape(1, OUT))


# speedup vs baseline: 26.5078x; 1.2506x over previous
import dataclasses
import functools

import jax
import jax.numpy as jnp
from jax import lax
from jax.experimental import pallas as pl
from jax.experimental.pallas import tpu as pltpu
from jax.experimental.pallas import tpu_sc as plsc

N = 10000
E = 320000
D = 128
G = 64
LIN = 128
OUT = 8

K = 128
NCHUNK = E // K
CPT = 80
HCPT = 40
NCPAD = 32 * CPT
NCLAST = NCHUNK - 31 * CPT
NTILES = 32
RPT = 624
RTAIL = N - 16 * RPT
HR = 640

_mesh = plsc.VectorSubcoreMesh(core_axis_name="c", subcore_axis_name="s")

_sc_params = pltpu.CompilerParams()
if "needs_layout_passes" in pltpu.CompilerParams.__dataclass_fields__:
    _sc_params = dataclasses.replace(_sc_params, needs_layout_passes=False)


@functools.partial(
    pl.kernel,
    out_type=jax.ShapeDtypeStruct((2, HR, D), jnp.float32),
    mesh=_mesh,
    scratch_types=[
        pltpu.VMEM((CPT, K), jnp.int32),
        pltpu.VMEM((HR, D), jnp.float32),
        pltpu.VMEM((K,), jnp.int32),
        pltpu.VMEM_SHARED((HR, D), jnp.float32),
    ],
    compiler_params=_sc_params,
)
def _sc_hist(dst_hbm, zeros_hbm, out_hbm, dstb, hist, iota_v, acc):
    core = lax.axis_index("c")
    sid = lax.axis_index("s")
    wid = sid * 2 + core
    nc = lax.select(wid == 31, NCLAST, CPT)
    hrt = HR // 16
    r0 = sid * hrt
    pltpu.sync_copy(dst_hbm.at[pl.ds(wid * CPT, CPT)], dstb)
    pltpu.sync_copy(zeros_hbm, hist)
    pltpu.sync_copy(zeros_hbm.at[pl.ds(r0, hrt)], acc.at[pl.ds(r0, hrt)])
    base16 = lax.iota(jnp.int32, 16)
    one16 = jnp.full((16,), 1.0, jnp.float32)

    @pl.loop(0, nc)
    def _(i):
        for j in range(K // 16):
            d16 = dstb[i, pl.ds(j * 16, 16)]
            row = lax.shift_right_logical(d16, 7)
            col = lax.bitwise_and(d16, 127)
            plsc.addupdate_scatter(hist, [row, col], one16)

    plsc.subcore_barrier()

    for k in range(HR // K):
        for j in range(K // 16):
            iota_v[pl.ds(j * 16, 16)] = base16 + (k * K + j * 16)
        pltpu.sync_copy(hist.at[pl.ds(k * K, K)], acc.at[iota_v], add=True)

    plsc.subcore_barrier()
    pltpu.sync_copy(acc.at[pl.ds(r0, hrt)], out_hbm.at[core].at[pl.ds(r0, hrt)])


@functools.partial(
    pl.kernel,
    out_type=jax.ShapeDtypeStruct((2, N, D), jnp.float32),
    mesh=_mesh,
    scratch_types=[
        pltpu.VMEM((HCPT, K), jnp.int32),
        pltpu.VMEM((HCPT, K), jnp.int32),
        pltpu.VMEM((K, D), jnp.float32),
        pltpu.VMEM((K, D), jnp.float32),
        pltpu.VMEM((K,), jnp.int32),
        pltpu.VMEM((K,), jnp.int32),
        pltpu.VMEM_SHARED((N, D), jnp.float32),
        pltpu.SemaphoreType.DMA,
        pltpu.SemaphoreType.DMA,
    ],
)
def _sc_agg(y_hbm, zeros_hbm, src_hbm, dst_hbm, out_hbm, srcb, dstb,
            rows0, rows1, didx0, didx1, acc, ssem0, ssem1):
    core = lax.axis_index("c")
    sid = lax.axis_index("s")
    wid = sid * 2 + core
    nc = lax.select(wid == 31, NCLAST, CPT)
    r0 = sid * RPT

    @pl.when(core == 0)
    def _():
        pltpu.sync_copy(y_hbm.at[pl.ds(r0, RPT)], acc.at[pl.ds(r0, RPT)])

        @pl.when(sid == 15)
        def _():
            pltpu.sync_copy(y_hbm.at[pl.ds(16 * RPT, RTAIL)],
                            acc.at[pl.ds(16 * RPT, RTAIL)])

    @pl.when(core == 1)
    def _():
        pltpu.sync_copy(zeros_hbm.at[pl.ds(r0, RPT)], acc.at[pl.ds(r0, RPT)])

        @pl.when(sid == 15)
        def _():
            pltpu.sync_copy(zeros_hbm.at[pl.ds(16 * RPT, RTAIL)],
                            acc.at[pl.ds(16 * RPT, RTAIL)])

    plsc.subcore_barrier()

    def _chunk(i, rows, didx, ssem, first):
        @pl.when(jnp.logical_not(first))
        def _():
            pltpu.make_async_copy(rows, acc.at[didx], ssem).wait()

        pltpu.sync_copy(y_hbm.at[srcb.at[i]], rows)
        for j in range(K // 16):
            didx[pl.ds(j * 16, 16)] = dstb[i, pl.ds(j * 16, 16)]
        pltpu.async_copy(rows, acc.at[didx], ssem, add=True)

    for p in range(CPT // HCPT):
        ncp = jnp.clip(nc - p * HCPT, 0, HCPT)

        @pl.when(ncp > 0)
        def _():
            pltpu.sync_copy(src_hbm.at[pl.ds(wid * CPT + p * HCPT, HCPT)], srcb)
            pltpu.sync_copy(dst_hbm.at[pl.ds(wid * CPT + p * HCPT, HCPT)], dstb)

            @pl.loop(0, ncp, step=2)
            def _(i):
                first = (i == 0) if p == 0 else jnp.bool_(False)
                _chunk(i, rows0, didx0, ssem0, first)
                _chunk(i + 1, rows1, didx1, ssem1, first)

    pltpu.make_async_copy(rows0, acc.at[didx0], ssem0).wait()
    pltpu.make_async_copy(rows1, acc.at[didx1], ssem1).wait()
    plsc.subcore_barrier()
    pltpu.sync_copy(acc.at[pl.ds(r0, RPT)], out_hbm.at[core].at[pl.ds(r0, RPT)])

    @pl.when(sid == 15)
    def _():
        pltpu.sync_copy(acc.at[pl.ds(16 * RPT, RTAIL)],
                        out_hbm.at[core].at[pl.ds(16 * RPT, RTAIL)])


R = 1000
GRID = N // R


def _tdinv_body(degp_ref, dinv_ref):
    dinv_ref[...] = lax.rsqrt(1.0 + degp_ref[0] + degp_ref[1])


_tdinv = pl.pallas_call(
    _tdinv_body,
    grid=(1,),
    in_specs=[pl.BlockSpec((2, HR, D), lambda i: (0, 0, 0))],
    out_specs=pl.BlockSpec((HR, D), lambda i: (0, 0)),
    out_shape=jax.ShapeDtypeStruct((HR, D), jnp.float32),
)


def _t1_body(x_ref, w_ref, dinv_ref, y_ref):
    dinv = dinv_ref[...]
    xw = jnp.dot(x_ref[...], w_ref[...], preferred_element_type=jnp.float32)
    y_ref[...] = xw * dinv


_t1 = pl.pallas_call(
    _t1_body,
    grid=(GRID,),
    in_specs=[
        pl.BlockSpec((R, D), lambda i: (i, 0)),
        pl.BlockSpec((D, D), lambda i: (0, 0)),
        pl.BlockSpec((R, 1), lambda i: (i, 0)),
    ],
    out_specs=pl.BlockSpec((R, D), lambda i: (i, 0)),
    out_shape=jax.ShapeDtypeStruct((N, D), jnp.float32),
)


def _t2_body(agg_ref, dinv_ref, b1_ref, w2_ref, y2_ref):
    dinv = dinv_ref[...]
    a = agg_ref[0] + agg_ref[1]
    h = jnp.maximum(a * dinv + b1_ref[...], 0.0)
    y2_ref[...] = jnp.dot(h, w2_ref[...], preferred_element_type=jnp.float32) * dinv


_t2 = pl.pallas_call(
    _t2_body,
    grid=(GRID,),
    in_specs=[
        pl.BlockSpec((2, R, D), lambda i: (0, i, 0)),
        pl.BlockSpec((R, 1), lambda i: (i, 0)),
        pl.BlockSpec((1, D), lambda i: (0, 0)),
        pl.BlockSpec((D, D), lambda i: (0, 0)),
    ],
    out_specs=pl.BlockSpec((R, D), lambda i: (i, 0)),
    out_shape=jax.ShapeDtypeStruct((N, D), jnp.float32),
)


def _t3_body(agg_ref, dinv_ref, b2_ref, batch_ref, lw_ref, lb_ref, ow_ref,
             ob_ref, out_ref, pool_ref):
    i = pl.program_id(0)

    @pl.when(i == 0)
    def _():
        pool_ref[...] = jnp.full((G, D), -jnp.inf, jnp.float32)

    dinv = dinv_ref[...]
    a = agg_ref[0] + agg_ref[1]
    h = jnp.maximum(a * dinv + b2_ref[...], 0.0)
    bidx = batch_ref[...]

    g_lo = bidx[0, 0]
    g_hi = bidx[R - 1, 0]

    def body(g, carry):
        vals = jnp.where(bidx == g, h, -jnp.inf)
        cur = pool_ref[pl.ds(g, 1), :]
        pool_ref[pl.ds(g, 1), :] = jnp.maximum(cur, vals.max(axis=0, keepdims=True))
        return carry

    lax.fori_loop(g_lo, g_hi + 1, body, 0)

    @pl.when(i == GRID - 1)
    def _():
        t = jnp.dot(pool_ref[...], lw_ref[...],
                    preferred_element_type=jnp.float32) + lb_ref[...]
        out_ref[...] = jnp.dot(t, ow_ref[...],
                               preferred_element_type=jnp.float32) + ob_ref[...]


_t3 = pl.pallas_call(
    _t3_body,
    grid=(GRID,),
    in_specs=[
        pl.BlockSpec((2, R, D), lambda i: (0, i, 0)),
        pl.BlockSpec((R, 1), lambda i: (i, 0)),
        pl.BlockSpec((1, D), lambda i: (0, 0)),
        pl.BlockSpec((R, 1), lambda i: (i, 0)),
        pl.BlockSpec((D, LIN), lambda i: (0, 0)),
        pl.BlockSpec((1, LIN), lambda i: (0, 0)),
        pl.BlockSpec((LIN, OUT), lambda i: (0, 0)),
        pl.BlockSpec((1, OUT), lambda i: (0, 0)),
    ],
    out_specs=pl.BlockSpec((G, OUT), lambda i: (0, 0)),
    out_shape=jax.ShapeDtypeStruct((G, OUT), jnp.float32),
    scratch_shapes=[pltpu.VMEM((G, D), jnp.float32)],
)


def kernel(x, edge_index, batch, W1, b1, W2, b2, lin_W, lin_b, out_W, out_b):
    pad = jnp.zeros((NCPAD - NCHUNK, K), jnp.int32)
    src2 = jnp.concatenate([edge_index[0].reshape(NCHUNK, K), pad])
    dst2 = jnp.concatenate([edge_index[1].reshape(NCHUNK, K), pad])
    zeros_nd = jnp.zeros((N, D), jnp.float32)
    zeros_hd = jnp.zeros((HR, D), jnp.float32)

    degp = _sc_hist(dst2, zeros_hd)
    dinv = _tdinv(degp).reshape(HR * D)[:N].reshape(N, 1)
    y1 = _t1(x, W1, dinv)
    agg1 = _sc_agg(y1, zeros_nd, src2, dst2)
    y2 = _t2(agg1, dinv, b1.reshape(1, D), W2)
    agg2 = _sc_agg(y2, zeros_nd, src2, dst2)
    return _t3(agg2, dinv, b2.reshape(1, D), batch.reshape(N, 1),
               lin_W, lin_b.reshape(1, LIN), out_W, out_b.reshape(1, OUT))


# speedup vs baseline: 30.2353x; 1.1406x over previous
import dataclasses
import functools

import jax
import jax.numpy as jnp
from jax import lax
from jax.experimental import pallas as pl
from jax.experimental.pallas import tpu as pltpu
from jax.experimental.pallas import tpu_sc as plsc

N = 10000
E = 320000
D = 128
G = 64
LIN = 128
OUT = 8

K = 128
NCHUNK = E // K
CPT = 80
HCPT = 40
NCPAD = 32 * CPT
NCLAST = NCHUNK - 31 * CPT
NTILES = 32
RPT = 624
RTAIL = N - 16 * RPT
HR = 640

_mesh = plsc.VectorSubcoreMesh(core_axis_name="c", subcore_axis_name="s")

_sc_params = pltpu.CompilerParams()
if "needs_layout_passes" in pltpu.CompilerParams.__dataclass_fields__:
    _sc_params = dataclasses.replace(_sc_params, needs_layout_passes=False)


@functools.partial(
    pl.kernel,
    out_type=jax.ShapeDtypeStruct((2, HR, D), jnp.float32),
    mesh=_mesh,
    scratch_types=[
        pltpu.VMEM((CPT, K), jnp.int32),
        pltpu.VMEM((HR, D), jnp.float32),
        pltpu.VMEM((K,), jnp.int32),
        pltpu.VMEM_SHARED((HR, D), jnp.float32),
    ],
    compiler_params=_sc_params,
)
def _sc_hist(dst_hbm, zeros_hbm, out_hbm, dstb, hist, iota_v, acc):
    core = lax.axis_index("c")
    sid = lax.axis_index("s")
    wid = sid * 2 + core
    nc = lax.select(wid == 31, NCLAST, CPT)
    hrt = HR // 16
    r0 = sid * hrt
    pltpu.sync_copy(dst_hbm.at[pl.ds(wid * CPT, CPT)], dstb)
    pltpu.sync_copy(zeros_hbm, hist)
    pltpu.sync_copy(zeros_hbm.at[pl.ds(r0, hrt)], acc.at[pl.ds(r0, hrt)])
    base16 = lax.iota(jnp.int32, 16)
    one16 = jnp.full((16,), 1.0, jnp.float32)

    @pl.loop(0, nc)
    def _(i):
        for j in range(K // 16):
            d16 = dstb[i, pl.ds(j * 16, 16)]
            row = lax.shift_right_logical(d16, 7)
            col = lax.bitwise_and(d16, 127)
            plsc.addupdate_scatter(hist, [row, col], one16)

    plsc.subcore_barrier()

    for k in range(HR // K):
        for j in range(K // 16):
            iota_v[pl.ds(j * 16, 16)] = base16 + (k * K + j * 16)
        pltpu.sync_copy(hist.at[pl.ds(k * K, K)], acc.at[iota_v], add=True)

    plsc.subcore_barrier()
    pltpu.sync_copy(acc.at[pl.ds(r0, hrt)], out_hbm.at[core].at[pl.ds(r0, hrt)])


@functools.partial(
    pl.kernel,
    out_type=jax.ShapeDtypeStruct((2, N, D), jnp.float32),
    mesh=_mesh,
    scratch_types=[
        pltpu.VMEM((HCPT, K), jnp.int32),
        pltpu.VMEM((HCPT, K), jnp.int32),
        pltpu.VMEM((K, D), jnp.float32),
        pltpu.VMEM((K, D), jnp.float32),
        pltpu.VMEM((K,), jnp.int32),
        pltpu.VMEM((K,), jnp.int32),
        pltpu.VMEM_SHARED((N, D), jnp.float32),
        pltpu.SemaphoreType.DMA,
        pltpu.SemaphoreType.DMA,
        pltpu.SemaphoreType.DMA,
        pltpu.SemaphoreType.DMA,
    ],
)
def _sc_agg(y_hbm, zeros_hbm, src_hbm, dst_hbm, out_hbm, srcb, dstb,
            rows0, rows1, didx0, didx1, acc, ssem0, ssem1, gsem0, gsem1):
    core = lax.axis_index("c")
    sid = lax.axis_index("s")
    wid = sid * 2 + core
    nc = lax.select(wid == 31, NCLAST, CPT)
    r0 = sid * RPT

    @pl.when(core == 0)
    def _():
        pltpu.sync_copy(y_hbm.at[pl.ds(r0, RPT)], acc.at[pl.ds(r0, RPT)])

        @pl.when(sid == 15)
        def _():
            pltpu.sync_copy(y_hbm.at[pl.ds(16 * RPT, RTAIL)],
                            acc.at[pl.ds(16 * RPT, RTAIL)])

    @pl.when(core == 1)
    def _():
        pltpu.sync_copy(zeros_hbm.at[pl.ds(r0, RPT)], acc.at[pl.ds(r0, RPT)])

        @pl.when(sid == 15)
        def _():
            pltpu.sync_copy(zeros_hbm.at[pl.ds(16 * RPT, RTAIL)],
                            acc.at[pl.ds(16 * RPT, RTAIL)])

    plsc.subcore_barrier()

    def _half(i, ncp, rows, didx, gsem, ssem):
        pltpu.make_async_copy(y_hbm.at[srcb.at[i]], rows, gsem).wait()
        for j in range(K // 16):
            didx[pl.ds(j * 16, 16)] = dstb[i, pl.ds(j * 16, 16)]
        pltpu.async_copy(rows, acc.at[didx], ssem, add=True)

        @pl.when(i + 2 < ncp)
        def _():
            pltpu.make_async_copy(rows, acc.at[didx], ssem).wait()
            pltpu.async_copy(y_hbm.at[srcb.at[i + 2]], rows, gsem)

    for p in range(CPT // HCPT):
        ncp = jnp.clip(nc - p * HCPT, 0, HCPT)

        @pl.when(ncp > 0)
        def _():
            pltpu.sync_copy(src_hbm.at[pl.ds(wid * CPT + p * HCPT, HCPT)], srcb)
            pltpu.sync_copy(dst_hbm.at[pl.ds(wid * CPT + p * HCPT, HCPT)], dstb)
            if p > 0:
                pltpu.make_async_copy(rows0, acc.at[didx0], ssem0).wait()
                pltpu.make_async_copy(rows1, acc.at[didx1], ssem1).wait()
            pltpu.async_copy(y_hbm.at[srcb.at[0]], rows0, gsem0)
            pltpu.async_copy(y_hbm.at[srcb.at[1]], rows1, gsem1)

            @pl.loop(0, ncp, step=2)
            def _(i):
                _half(i, ncp, rows0, didx0, gsem0, ssem0)
                _half(i + 1, ncp, rows1, didx1, gsem1, ssem1)

    pltpu.make_async_copy(rows0, acc.at[didx0], ssem0).wait()
    pltpu.make_async_copy(rows1, acc.at[didx1], ssem1).wait()
    plsc.subcore_barrier()
    pltpu.sync_copy(acc.at[pl.ds(r0, RPT)], out_hbm.at[core].at[pl.ds(r0, RPT)])

    @pl.when(sid == 15)
    def _():
        pltpu.sync_copy(acc.at[pl.ds(16 * RPT, RTAIL)],
                        out_hbm.at[core].at[pl.ds(16 * RPT, RTAIL)])


R = 1000
GRID = N // R


def _tdinv_body(degp_ref, dinv_ref):
    dinv_ref[...] = lax.rsqrt(1.0 + degp_ref[0] + degp_ref[1])


_tdinv = pl.pallas_call(
    _tdinv_body,
    grid=(1,),
    in_specs=[pl.BlockSpec((2, HR, D), lambda i: (0, 0, 0))],
    out_specs=pl.BlockSpec((HR, D), lambda i: (0, 0)),
    out_shape=jax.ShapeDtypeStruct((HR, D), jnp.float32),
)


def _t1_body(x_ref, w_ref, dinv_ref, y_ref):
    dinv = dinv_ref[...]
    xw = jnp.dot(x_ref[...], w_ref[...], preferred_element_type=jnp.float32)
    y_ref[...] = xw * dinv


_t1 = pl.pallas_call(
    _t1_body,
    grid=(GRID,),
    in_specs=[
        pl.BlockSpec((R, D), lambda i: (i, 0)),
        pl.BlockSpec((D, D), lambda i: (0, 0)),
        pl.BlockSpec((R, 1), lambda i: (i, 0)),
    ],
    out_specs=pl.BlockSpec((R, D), lambda i: (i, 0)),
    out_shape=jax.ShapeDtypeStruct((N, D), jnp.float32),
)


def _t2_body(agg_ref, dinv_ref, b1_ref, w2_ref, y2_ref):
    dinv = dinv_ref[...]
    a = agg_ref[0] + agg_ref[1]
    h = jnp.maximum(a * dinv + b1_ref[...], 0.0)
    y2_ref[...] = jnp.dot(h, w2_ref[...], preferred_element_type=jnp.float32) * dinv


_t2 = pl.pallas_call(
    _t2_body,
    grid=(GRID,),
    in_specs=[
        pl.BlockSpec((2, R, D), lambda i: (0, i, 0)),
        pl.BlockSpec((R, 1), lambda i: (i, 0)),
        pl.BlockSpec((1, D), lambda i: (0, 0)),
        pl.BlockSpec((D, D), lambda i: (0, 0)),
    ],
    out_specs=pl.BlockSpec((R, D), lambda i: (i, 0)),
    out_shape=jax.ShapeDtypeStruct((N, D), jnp.float32),
)


def _t3_body(agg_ref, dinv_ref, b2_ref, batch_ref, lw_ref, lb_ref, ow_ref,
             ob_ref, out_ref, pool_ref):
    i = pl.program_id(0)

    @pl.when(i == 0)
    def _():
        pool_ref[...] = jnp.full((G, D), -jnp.inf, jnp.float32)

    dinv = dinv_ref[...]
    a = agg_ref[0] + agg_ref[1]
    h = jnp.maximum(a * dinv + b2_ref[...], 0.0)
    bidx = batch_ref[...]

    g_lo = bidx[0, 0]
    g_hi = bidx[R - 1, 0]

    def body(g, carry):
        vals = jnp.where(bidx == g, h, -jnp.inf)
        cur = pool_ref[pl.ds(g, 1), :]
        pool_ref[pl.ds(g, 1), :] = jnp.maximum(cur, vals.max(axis=0, keepdims=True))
        return carry

    lax.fori_loop(g_lo, g_hi + 1, body, 0)

    @pl.when(i == GRID - 1)
    def _():
        t = jnp.dot(pool_ref[...], lw_ref[...],
                    preferred_element_type=jnp.float32) + lb_ref[...]
        out_ref[...] = jnp.dot(t, ow_ref[...],
                               preferred_element_type=jnp.float32) + ob_ref[...]


_t3 = pl.pallas_call(
    _t3_body,
    grid=(GRID,),
    in_specs=[
        pl.BlockSpec((2, R, D), lambda i: (0, i, 0)),
        pl.BlockSpec((R, 1), lambda i: (i, 0)),
        pl.BlockSpec((1, D), lambda i: (0, 0)),
        pl.BlockSpec((R, 1), lambda i: (i, 0)),
        pl.BlockSpec((D, LIN), lambda i: (0, 0)),
        pl.BlockSpec((1, LIN), lambda i: (0, 0)),
        pl.BlockSpec((LIN, OUT), lambda i: (0, 0)),
        pl.BlockSpec((1, OUT), lambda i: (0, 0)),
    ],
    out_specs=pl.BlockSpec((G, OUT), lambda i: (0, 0)),
    out_shape=jax.ShapeDtypeStruct((G, OUT), jnp.float32),
    scratch_shapes=[pltpu.VMEM((G, D), jnp.float32)],
)


def kernel(x, edge_index, batch, W1, b1, W2, b2, lin_W, lin_b, out_W, out_b):
    pad = jnp.zeros((NCPAD - NCHUNK, K), jnp.int32)
    src2 = jnp.concatenate([edge_index[0].reshape(NCHUNK, K), pad])
    dst2 = jnp.concatenate([edge_index[1].reshape(NCHUNK, K), pad])
    zeros_nd = jnp.zeros((N, D), jnp.float32)
    zeros_hd = jnp.zeros((HR, D), jnp.float32)

    degp = _sc_hist(dst2, zeros_hd)
    dinv = _tdinv(degp).reshape(HR * D)[:N].reshape(N, 1)
    y1 = _t1(x, W1, dinv)
    agg1 = _sc_agg(y1, zeros_nd, src2, dst2)
    y2 = _t2(agg1, dinv, b1.reshape(1, D), W2)
    agg2 = _sc_agg(y2, zeros_nd, src2, dst2)
    return _t3(agg2, dinv, b2.reshape(1, D), batch.reshape(N, 1),
               lin_W, lin_b.reshape(1, LIN), out_W, out_b.reshape(1, OUT))
